# Initial kernel scaffold; baseline (speedup 1.0000x reference)
import jax
import jax.numpy as jnp
from jax.experimental import pallas as pl


def kernel(feat, edge_index, W0, W1, a0, a1):
    raise NotImplementedError("write your pallas kernel here")


# speedup vs baseline: 6.3320x; 6.3320x over previous
import functools

import jax
import jax.numpy as jnp
from jax import lax
from jax.experimental import pallas as pl
from jax.experimental.pallas import tpu as pltpu, tpu_sc as plsc

N = 10000
E = 320000
D = 128
DH = D // 2

NC = 2
NS = 16
NW = NC * NS

K = 128
EPT = E // NS
ECH = (EPT + K - 1) // K
EPAD = NS * ECH * K

RPT = 632
NROWS = RPT * NS
DUMP = N

OFF = NROWS
DT_ROWS = 2 * NROWS
DCH = (2 * E // NW + K - 1) // K
DPAD = NW * DCH * K
DDUMP = DT_ROWS - 1
DRPT = DT_ROWS // NS
CW = 8

_mesh = plsc.VectorSubcoreMesh(core_axis_name="c", subcore_axis_name="s")
_sc_params = pltpu.CompilerParams(use_tc_tiling_on_sc=False)


@functools.partial(
    pl.kernel,
    out_type=jax.ShapeDtypeStruct((NC, DT_ROWS, CW), jnp.float32),
    mesh=_mesh,
    compiler_params=_sc_params,
    scratch_types=[
        pltpu.VMEM((DCH, K), jnp.int32),
        pltpu.VMEM((K, CW), jnp.float32),
        pltpu.VMEM((DRPT, CW), jnp.float32),
        pltpu.VMEM_SHARED((DT_ROWS, CW), jnp.float32),
    ],
)
def _deg_kernel(comb_hbm, ones_hbm, zeros_hbm, out_hbm, idx_v, ones_v, z_v, table):
    c = lax.axis_index("c")
    s = lax.axis_index("s")
    t = c * NS + s

    pltpu.sync_copy(comb_hbm.at[t], idx_v)
    pltpu.sync_copy(ones_hbm, ones_v)
    pltpu.sync_copy(zeros_hbm, z_v)

    pltpu.sync_copy(z_v, table.at[pl.ds(s * DRPT, DRPT)])
    plsc.subcore_barrier()

    def body(ch, carry):
        pltpu.sync_copy(ones_v, table.at[idx_v.at[ch]], add=True)
        return carry

    lax.fori_loop(0, DCH, body, 0)
    plsc.subcore_barrier()

    pltpu.sync_copy(table.at[pl.ds(s * DRPT, DRPT)], z_v)
    pltpu.sync_copy(z_v, out_hbm.at[c, pl.ds(s * DRPT, DRPT)])


@functools.partial(
    pl.kernel,
    out_type=jax.ShapeDtypeStruct((NC, NROWS, DH), jnp.float32),
    mesh=_mesh,
    compiler_params=_sc_params,
    scratch_types=[
        pltpu.VMEM((ECH, K), jnp.int32),
        pltpu.VMEM((ECH, K), jnp.int32),
        pltpu.VMEM((K, DH), jnp.float32),
        pltpu.VMEM((K, DH), jnp.float32),
        pltpu.VMEM_SHARED((NROWS, DH), jnp.float32),
        pltpu.SemaphoreType.DMA,
        pltpu.SemaphoreType.DMA,
    ],
)
def _edge_kernel(hs2_hbm, src_hbm, dst_hbm, zeros_hbm, out_hbm,
                 src_v, dst_v, rows_a, rows_b, acc, sem_a, sem_b):
    c = lax.axis_index("c")
    s = lax.axis_index("s")

    pltpu.sync_copy(src_hbm.at[c, s], src_v)
    pltpu.sync_copy(dst_hbm.at[s], dst_v)
    pltpu.sync_copy(zeros_hbm, rows_a)

    base = s * RPT
    for j in range(4):
        pltpu.sync_copy(rows_a, acc.at[pl.ds(base + j * K, K)])
    pltpu.sync_copy(rows_a.at[pl.ds(0, RPT - 4 * K)],
                    acc.at[pl.ds(base + 4 * K, RPT - 4 * K)])
    plsc.subcore_barrier()

    pltpu.async_copy(hs2_hbm.at[src_v.at[0]], rows_a, sem_a)

    def body(ch, carry):
        even = lax.rem(ch, 2) == 0

        @pl.when(jnp.logical_and(even, ch + 1 < ECH))
        def _():
            pltpu.async_copy(hs2_hbm.at[src_v.at[ch + 1]], rows_b, sem_b)

        @pl.when(jnp.logical_and(jnp.logical_not(even), ch + 1 < ECH))
        def _():
            pltpu.async_copy(hs2_hbm.at[src_v.at[ch + 1]], rows_a, sem_a)

        @pl.when(even)
        def _():
            pltpu.make_async_copy(hs2_hbm.at[src_v.at[0]], rows_a, sem_a).wait()
            pltpu.sync_copy(rows_a, acc.at[dst_v.at[ch]], add=True)

        @pl.when(jnp.logical_not(even))
        def _():
            pltpu.make_async_copy(hs2_hbm.at[src_v.at[0]], rows_b, sem_b).wait()
            pltpu.sync_copy(rows_b, acc.at[dst_v.at[ch]], add=True)

        return carry

    lax.fori_loop(0, ECH, body, 0)
    plsc.subcore_barrier()

    for j in range(4):
        pltpu.sync_copy(acc.at[pl.ds(base + j * K, K)], rows_a)
        pltpu.sync_copy(rows_a, out_hbm.at[c, pl.ds(base + j * K, K)])
    pltpu.sync_copy(acc.at[pl.ds(base + 4 * K, RPT - 4 * K)],
                    rows_a.at[pl.ds(0, RPT - 4 * K)])
    pltpu.sync_copy(rows_a.at[pl.ds(0, RPT - 4 * K)],
                    out_hbm.at[c, pl.ds(base + 4 * K, RPT - 4 * K)])


_RB = 1000


def _prep_body(d0s, d1s, d0d, d1d, feat, so_ref, si_ref, fs_ref):
    cs = d0s[...] + d1s[...]
    cd = d0d[...] + d1d[...]
    so = lax.rsqrt(jnp.maximum(cs, 1.0))
    si = lax.rsqrt(jnp.maximum(cd, 1.0))
    so_ref[...] = so
    si_ref[...] = si
    fs_ref[...] = feat[...] * so


@jax.jit
def _prep(d0s, d1s, d0d, d1d, feat):
    vec = pl.BlockSpec((_RB, 1), lambda i: (i, 0))
    return pl.pallas_call(
        _prep_body,
        grid=(N // _RB,),
        in_specs=[vec, vec, vec, vec, pl.BlockSpec((_RB, D), lambda i: (i, 0))],
        out_specs=[vec, vec, pl.BlockSpec((_RB, D), lambda i: (i, 0))],
        out_shape=[
            jax.ShapeDtypeStruct((N, 1), jnp.float32),
            jax.ShapeDtypeStruct((N, 1), jnp.float32),
            jax.ShapeDtypeStruct((N, D), jnp.float32),
        ],
    )(d0s, d1s, d0d, d1d, feat)


def _dense_body(parts, si, so, w, a, h_ref, hs_ref, pool_ref):
    i = pl.program_id(0)
    agg = jnp.concatenate([parts[0], parts[1]], axis=-1) * si[...]
    out = jnp.dot(agg, w[...], preferred_element_type=jnp.float32)
    aa = a[0, 0]
    h = jnp.where(out >= 0.0, out, aa * out)
    h_ref[...] = h
    hs_ref[...] = h * so[...]

    @pl.when(i == 0)
    def _():
        pool_ref[...] = jnp.zeros_like(pool_ref)

    pool_ref[...] += jnp.sum(h, axis=0, keepdims=True)


@jax.jit
def _dense(parts, si, so, w, a):
    vec = pl.BlockSpec((_RB, 1), lambda i: (i, 0))
    mat = pl.BlockSpec((_RB, D), lambda i: (i, 0))
    return pl.pallas_call(
        _dense_body,
        grid=(N // _RB,),
        in_specs=[
            pl.BlockSpec((NC, _RB, DH), lambda i: (0, i, 0)),
            vec, vec,
            pl.BlockSpec((D, D), lambda i: (0, 0)),
            pl.BlockSpec(memory_space=pltpu.SMEM),
        ],
        out_specs=[mat, mat, pl.BlockSpec((1, D), lambda i: (0, 0))],
        out_shape=[
            jax.ShapeDtypeStruct((N, D), jnp.float32),
            jax.ShapeDtypeStruct((N, D), jnp.float32),
            jax.ShapeDtypeStruct((1, D), jnp.float32),
        ],
    )(parts, si, so, w, a)


def kernel(feat, edge_index, W0, W1, a0, a1):
    src = edge_index[0]
    dst = edge_index[1]

    src2 = jnp.pad(2 * src, (0, EPAD - E)).reshape(NS, ECH, K)
    srci = jnp.stack([src2, src2 + 1])
    dstp = jnp.pad(dst, (0, EPAD - E), constant_values=DUMP).reshape(NS, ECH, K)

    comb = jnp.concatenate([src, dst + OFF])
    combp = jnp.pad(comb, (0, DPAD - 2 * E),
                    constant_values=DDUMP).reshape(NW, DCH, K)

    ones_cw = jnp.ones((K, CW), jnp.float32)
    zeros_cw = jnp.zeros((DRPT, CW), jnp.float32)
    zeros_kd = jnp.zeros((K, DH), jnp.float32)

    dcnt = _deg_kernel(combp, ones_cw, zeros_cw)
    d0s = dcnt[0, :N, 0:1]
    d1s = dcnt[1, :N, 0:1]
    d0d = dcnt[0, OFF:OFF + N, 0:1]
    d1d = dcnt[1, OFF:OFF + N, 0:1]

    s_out, s_in, fs = _prep(d0s, d1s, d0d, d1d, feat)

    a0_2d = a0.reshape(1, 1)
    a1_2d = a1.reshape(1, 1)

    parts1 = _edge_kernel(fs.reshape(2 * N, DH), srci, dstp, zeros_kd)
    h1, h1s, pool1 = _dense(parts1, s_in, s_out, W0, a0_2d)

    parts2 = _edge_kernel(h1s.reshape(2 * N, DH), srci, dstp, zeros_kd)
    h2, _h2s, pool2 = _dense(parts2, s_in, s_out, W1, a1_2d)

    hg = jnp.concatenate([pool1, pool2], axis=-1)
    return (h2, hg)


# speedup vs baseline: 7.5225x; 1.1880x over previous
import functools

import jax
import jax.numpy as jnp
from jax import lax
from jax.experimental import pallas as pl
from jax.experimental.pallas import tpu as pltpu, tpu_sc as plsc

N = 10000
E = 320000
D = 128
DH = D // 2

NC = 2
NS = 16
NW = NC * NS

K = 128
EPT = E // NS
ECH = (EPT + K - 1) // K
EPAD = NS * ECH * K

RPT = 632
NROWS = RPT * NS
DUMP = N

CW = 8
DRPT = NROWS // NS

_mesh = plsc.VectorSubcoreMesh(core_axis_name="c", subcore_axis_name="s")
_sc_params = pltpu.CompilerParams(use_tc_tiling_on_sc=False)


@functools.partial(
    pl.kernel,
    out_type=jax.ShapeDtypeStruct((NC, NROWS, CW), jnp.float32),
    mesh=_mesh,
    compiler_params=_sc_params,
    scratch_types=[
        pltpu.VMEM((ECH, K), jnp.int32),
        pltpu.VMEM((K, CW), jnp.float32),
        pltpu.VMEM((DRPT, CW), jnp.float32),
        pltpu.VMEM_SHARED((NROWS, CW), jnp.float32),
        pltpu.SemaphoreType.DMA,
    ],
)
def _deg_kernel(ei_hbm, ones_hbm, zeros_hbm, out_hbm, idx_v, ones_v, z_v,
                table, sem):
    c = lax.axis_index("c")
    s = lax.axis_index("s")

    idx_load = pltpu.async_copy(ei_hbm.at[c, s], idx_v, sem)
    pltpu.sync_copy(ones_hbm, ones_v)
    pltpu.sync_copy(zeros_hbm, z_v)

    pltpu.sync_copy(z_v, table.at[pl.ds(s * DRPT, DRPT)])
    idx_load.wait()
    plsc.subcore_barrier()

    def body(ch, carry):
        pltpu.sync_copy(ones_v, table.at[idx_v.at[ch]], add=True)
        return carry

    lax.fori_loop(0, ECH, body, 0)
    plsc.subcore_barrier()

    pltpu.sync_copy(table.at[pl.ds(s * DRPT, DRPT)], z_v)
    pltpu.sync_copy(z_v, out_hbm.at[c, pl.ds(s * DRPT, DRPT)])


@functools.partial(
    pl.kernel,
    out_type=jax.ShapeDtypeStruct((NC, NROWS, DH), jnp.float32),
    mesh=_mesh,
    compiler_params=_sc_params,
    scratch_types=[
        pltpu.VMEM((ECH, K), jnp.int32),
        pltpu.VMEM((ECH, K), jnp.int32),
        pltpu.VMEM((K, DH), jnp.float32),
        pltpu.VMEM((K, DH), jnp.float32),
        pltpu.VMEM((K, DH), jnp.float32),
        pltpu.VMEM_SHARED((NROWS, DH), jnp.float32),
        pltpu.SemaphoreType.DMA,
        pltpu.SemaphoreType.DMA,
        pltpu.SemaphoreType.DMA,
        pltpu.SemaphoreType.DMA,
        pltpu.SemaphoreType.DMA,
        pltpu.SemaphoreType.DMA,
        pltpu.SemaphoreType.DMA,
    ],
)
def _edge_kernel(hs2_hbm, ei_hbm, zeros_hbm, out_hbm,
                 src_v, dst_v, buf_a, buf_b, buf_c, acc,
                 ga, gb, gc, sa, sb, sc, isem):
    c = lax.axis_index("c")
    s = lax.axis_index("s")
    page = hs2_hbm.at[c]

    src_load = pltpu.async_copy(ei_hbm.at[0, s], src_v, isem)
    dst_load = pltpu.async_copy(ei_hbm.at[1, s], dst_v, isem)
    pltpu.sync_copy(zeros_hbm, buf_a)

    base = s * RPT
    for j in range(4):
        pltpu.sync_copy(buf_a, acc.at[pl.ds(base + j * K, K)])
    pltpu.sync_copy(buf_a.at[pl.ds(0, RPT - 4 * K)],
                    acc.at[pl.ds(base + 4 * K, RPT - 4 * K)])
    src_load.wait()
    dst_load.wait()
    plsc.subcore_barrier()

    def gwait(sem):
        pltpu.make_async_copy(page.at[src_v.at[0]], buf_a, sem).wait()

    def swait(sem):
        pltpu.make_async_copy(buf_a, acc.at[dst_v.at[0]], sem).wait()

    pltpu.async_copy(page.at[src_v.at[0]], buf_a, ga)
    pltpu.async_copy(page.at[src_v.at[1]], buf_b, gb)

    def body(i, carry):
        ch = 3 * i

        gwait(ga)
        pltpu.async_copy(buf_a, acc.at[dst_v.at[ch]], sa, add=True)

        @pl.when(i > 0)
        def _():
            swait(sc)

        pltpu.async_copy(page.at[src_v.at[ch + 2]], buf_c, gc)

        gwait(gb)
        pltpu.async_copy(buf_b, acc.at[dst_v.at[ch + 1]], sb, add=True)
        swait(sa)
        pltpu.async_copy(page.at[src_v.at[ch + 3]], buf_a, ga)

        gwait(gc)
        pltpu.async_copy(buf_c, acc.at[dst_v.at[ch + 2]], sc, add=True)
        swait(sb)

        @pl.when(i < 51)
        def _():
            pltpu.async_copy(page.at[src_v.at[ch + 4]], buf_b, gb)

        return carry

    lax.fori_loop(0, (ECH - 1) // 3, body, 0)

    gwait(ga)
    pltpu.async_copy(buf_a, acc.at[dst_v.at[ECH - 1]], sa, add=True)
    swait(sc)
    swait(sa)
    plsc.subcore_barrier()

    for j in range(4):
        pltpu.sync_copy(acc.at[pl.ds(base + j * K, K)], buf_a)
        pltpu.sync_copy(buf_a, out_hbm.at[c, pl.ds(base + j * K, K)])
    pltpu.sync_copy(acc.at[pl.ds(base + 4 * K, RPT - 4 * K)],
                    buf_a.at[pl.ds(0, RPT - 4 * K)])
    pltpu.sync_copy(buf_a.at[pl.ds(0, RPT - 4 * K)],
                    out_hbm.at[c, pl.ds(base + 4 * K, RPT - 4 * K)])


_RB = 1000


def _prep_body(ds, dd, feat, so_ref, si_ref, fs_ref):
    so = lax.rsqrt(jnp.maximum(ds[...], 1.0))
    si = lax.rsqrt(jnp.maximum(dd[...], 1.0))
    so_ref[...] = so
    si_ref[...] = si
    fs_ref[0] = feat[:, :DH] * so
    fs_ref[1] = feat[:, DH:] * so


@jax.jit
def _prep(ds, dd, feat):
    vec = pl.BlockSpec((_RB, 1), lambda i: (i, 0))
    return pl.pallas_call(
        _prep_body,
        grid=(N // _RB,),
        in_specs=[vec, vec, pl.BlockSpec((_RB, D), lambda i: (i, 0))],
        out_specs=[vec, vec, pl.BlockSpec((NC, _RB, DH), lambda i: (0, i, 0))],
        out_shape=[
            jax.ShapeDtypeStruct((N, 1), jnp.float32),
            jax.ShapeDtypeStruct((N, 1), jnp.float32),
            jax.ShapeDtypeStruct((NC, NROWS, DH), jnp.float32),
        ],
    )(ds, dd, feat)


def _dense_body(want_h, parts, si, so, w, a, out_ref, pool_ref):
    i = pl.program_id(0)
    agg = jnp.concatenate([parts[0], parts[1]], axis=-1) * si[...]
    out = jnp.dot(agg, w[...], preferred_element_type=jnp.float32)
    aa = a[0, 0]
    h = jnp.where(out >= 0.0, out, aa * out)
    if want_h:
        out_ref[...] = h
    else:
        hs = h * so[...]
        out_ref[0] = hs[:, :DH]
        out_ref[1] = hs[:, DH:]

    @pl.when(i == 0)
    def _():
        pool_ref[...] = jnp.zeros_like(pool_ref)

    pool_ref[...] += jnp.sum(h, axis=0, keepdims=True)


@functools.partial(jax.jit, static_argnums=0)
def _dense(want_h, parts, si, so, w, a):
    vec = pl.BlockSpec((_RB, 1), lambda i: (i, 0))
    if want_h:
        out_spec = pl.BlockSpec((_RB, D), lambda i: (i, 0))
        out_shape = jax.ShapeDtypeStruct((N, D), jnp.float32)
    else:
        out_spec = pl.BlockSpec((NC, _RB, DH), lambda i: (0, i, 0))
        out_shape = jax.ShapeDtypeStruct((NC, NROWS, DH), jnp.float32)
    return pl.pallas_call(
        functools.partial(_dense_body, want_h),
        grid=(N // _RB,),
        in_specs=[
            pl.BlockSpec((NC, _RB, DH), lambda i: (0, i, 0)),
            vec, vec,
            pl.BlockSpec((D, D), lambda i: (0, 0)),
            pl.BlockSpec(memory_space=pltpu.SMEM),
        ],
        out_specs=[out_spec, pl.BlockSpec((1, D), lambda i: (0, 0))],
        out_shape=[out_shape, jax.ShapeDtypeStruct((1, D), jnp.float32)],
    )(parts, si, so, w, a)


def kernel(feat, edge_index, W0, W1, a0, a1):
    ei_p = jnp.pad(edge_index, ((0, 0), (0, EPAD - E)),
                   constant_values=DUMP).reshape(2, NS, ECH, K)

    ones_cw = jnp.ones((K, CW), jnp.float32)
    zeros_cw = jnp.zeros((DRPT, CW), jnp.float32)
    zeros_kd = jnp.zeros((K, DH), jnp.float32)

    dcnt = _deg_kernel(ei_p, ones_cw, zeros_cw)
    s_out, s_in, fs2 = _prep(dcnt[0, :N, 0:1], dcnt[1, :N, 0:1], feat)

    a0_2d = a0.reshape(1, 1)
    a1_2d = a1.reshape(1, 1)

    parts1 = _edge_kernel(fs2, ei_p, zeros_kd)
    hs2, pool1 = _dense(False, parts1, s_in, s_out, W0, a0_2d)

    parts2 = _edge_kernel(hs2, ei_p, zeros_kd)
    h2, pool2 = _dense(True, parts2, s_in, s_out, W1, a1_2d)

    hg = jnp.concatenate([pool1, pool2], axis=-1)
    return (h2, hg)


# speedup vs baseline: 7.7308x; 1.0277x over previous
import functools

import jax
import jax.numpy as jnp
from jax import lax
from jax.experimental import pallas as pl
from jax.experimental.pallas import tpu as pltpu, tpu_sc as plsc

N = 10000
E = 320000
D = 128
DH = D // 2

NC = 2
NS = 16
NW = NC * NS

K = 128
EPT = E // NS
ECH = (EPT + K - 1) // K
EPAD = NS * ECH * K

RPT = 632
NROWS = RPT * NS
DUMP = N

CW = 8
DRPT = NROWS // NS

_mesh = plsc.VectorSubcoreMesh(core_axis_name="c", subcore_axis_name="s")
_sc_params = pltpu.CompilerParams(use_tc_tiling_on_sc=False)


@functools.partial(
    pl.kernel,
    out_type=jax.ShapeDtypeStruct((NC, NROWS, CW), jnp.float32),
    mesh=_mesh,
    compiler_params=_sc_params,
    scratch_types=[
        pltpu.VMEM((ECH, K), jnp.int32),
        pltpu.VMEM((K, CW), jnp.float32),
        pltpu.VMEM((DRPT, CW), jnp.float32),
        pltpu.VMEM_SHARED((NROWS, CW), jnp.float32),
        pltpu.SemaphoreType.DMA,
    ],
)
def _deg_kernel(ei_hbm, ones_hbm, zeros_hbm, out_hbm, idx_v, ones_v, z_v,
                table, sem):
    c = lax.axis_index("c")
    s = lax.axis_index("s")

    idx_load = pltpu.async_copy(ei_hbm.at[c, s], idx_v, sem)
    pltpu.sync_copy(ones_hbm, ones_v)
    pltpu.sync_copy(zeros_hbm, z_v)

    pltpu.sync_copy(z_v, table.at[pl.ds(s * DRPT, DRPT)])
    idx_load.wait()
    plsc.subcore_barrier()

    def body(ch, carry):
        pltpu.sync_copy(ones_v, table.at[idx_v.at[ch]], add=True)
        return carry

    lax.fori_loop(0, ECH, body, 0)
    plsc.subcore_barrier()

    pltpu.sync_copy(table.at[pl.ds(s * DRPT, DRPT)], z_v)
    pltpu.sync_copy(z_v, out_hbm.at[c, pl.ds(s * DRPT, DRPT)])


@functools.partial(
    pl.kernel,
    out_type=jax.ShapeDtypeStruct((NC, NROWS, DH), jnp.float32),
    mesh=_mesh,
    compiler_params=_sc_params,
    scratch_types=[
        pltpu.VMEM((ECH, K), jnp.int32),
        pltpu.VMEM((ECH, K), jnp.int32),
        pltpu.VMEM((K, DH), jnp.float32),
        pltpu.VMEM((K, DH), jnp.float32),
        pltpu.VMEM((K, DH), jnp.float32),
        pltpu.VMEM((K, DH), jnp.float32),
        pltpu.VMEM((K, DH), jnp.float32),
        pltpu.VMEM_SHARED((NROWS, DH), jnp.float32),
        pltpu.SemaphoreType.DMA,
        pltpu.SemaphoreType.DMA,
        pltpu.SemaphoreType.DMA,
        pltpu.SemaphoreType.DMA,
        pltpu.SemaphoreType.DMA,
        pltpu.SemaphoreType.DMA,
        pltpu.SemaphoreType.DMA,
        pltpu.SemaphoreType.DMA,
        pltpu.SemaphoreType.DMA,
        pltpu.SemaphoreType.DMA,
        pltpu.SemaphoreType.DMA,
    ],
)
def _edge_kernel(hs2_hbm, ei_hbm, zeros_hbm, out_hbm,
                 src_v, dst_v, b0, b1, b2, b3, b4, acc,
                 g0, g1, g2, g3, g4, s0, s1, s2, s3, s4, isem):
    c = lax.axis_index("c")
    s = lax.axis_index("s")
    page = hs2_hbm.at[c]
    bufs = (b0, b1, b2, b3, b4)
    gsem = (g0, g1, g2, g3, g4)
    ssem = (s0, s1, s2, s3, s4)

    src_load = pltpu.async_copy(ei_hbm.at[0, s], src_v, isem)
    dst_load = pltpu.async_copy(ei_hbm.at[1, s], dst_v, isem)
    pltpu.sync_copy(zeros_hbm, b0)

    base = s * RPT
    for j in range(4):
        pltpu.sync_copy(b0, acc.at[pl.ds(base + j * K, K)])
    pltpu.sync_copy(b0.at[pl.ds(0, RPT - 4 * K)],
                    acc.at[pl.ds(base + 4 * K, RPT - 4 * K)])
    src_load.wait()
    dst_load.wait()
    plsc.subcore_barrier()

    def issue_g(ch, b):
        pltpu.async_copy(page.at[src_v.at[ch]], bufs[b], gsem[b])

    def issue_s(ch, b):
        pltpu.async_copy(bufs[b], acc.at[dst_v.at[ch]], ssem[b], add=True)

    def gwait(b):
        pltpu.make_async_copy(page.at[src_v.at[0]], bufs[b], gsem[b]).wait()

    def swait(b):
        pltpu.make_async_copy(bufs[b], acc.at[dst_v.at[0]], ssem[b]).wait()

    for ch in range(4):
        issue_g(ch, ch)

    gwait(0); issue_s(0, 0); issue_g(4, 4)
    gwait(1); issue_s(1, 1); swait(0); issue_g(5, 0)
    gwait(2); issue_s(2, 2); swait(1); issue_g(6, 1)
    gwait(3); issue_s(3, 3); swait(2); issue_g(7, 2)
    gwait(4); issue_s(4, 4); swait(3); issue_g(8, 3)

    def body(i, carry):
        ch = 5 * i
        gwait(0); issue_s(ch, 0); swait(4); issue_g(ch + 4, 4)
        gwait(1); issue_s(ch + 1, 1); swait(0); issue_g(ch + 5, 0)
        gwait(2); issue_s(ch + 2, 2); swait(1); issue_g(ch + 6, 1)
        gwait(3); issue_s(ch + 3, 3); swait(2); issue_g(ch + 7, 2)
        gwait(4); issue_s(ch + 4, 4); swait(3); issue_g(ch + 8, 3)
        return carry

    lax.fori_loop(1, 30, body, 0)

    gwait(0); issue_s(150, 0); swait(4); issue_g(154, 4)
    gwait(1); issue_s(151, 1); swait(0); issue_g(155, 0)
    gwait(2); issue_s(152, 2); swait(1); issue_g(156, 1)
    gwait(3); issue_s(153, 3); swait(2)
    gwait(4); issue_s(154, 4); swait(3)
    gwait(0); issue_s(155, 0); swait(4)
    gwait(1); issue_s(156, 1); swait(0)
    swait(1)
    plsc.subcore_barrier()

    pltpu.sync_copy(acc.at[pl.ds(base, RPT)], out_hbm.at[c, pl.ds(base, RPT)])


_RB = 1000


def _prep_body(ds, dd, feat, so_ref, si_ref, fs_ref):
    so = lax.rsqrt(jnp.maximum(ds[...], 1.0))
    si = lax.rsqrt(jnp.maximum(dd[...], 1.0))
    so_ref[...] = so
    si_ref[...] = si
    fs_ref[0] = feat[:, :DH] * so
    fs_ref[1] = feat[:, DH:] * so


@jax.jit
def _prep(ds, dd, feat):
    vec = pl.BlockSpec((_RB, 1), lambda i: (i, 0))
    return pl.pallas_call(
        _prep_body,
        grid=(N // _RB,),
        in_specs=[vec, vec, pl.BlockSpec((_RB, D), lambda i: (i, 0))],
        out_specs=[vec, vec, pl.BlockSpec((NC, _RB, DH), lambda i: (0, i, 0))],
        out_shape=[
            jax.ShapeDtypeStruct((N, 1), jnp.float32),
            jax.ShapeDtypeStruct((N, 1), jnp.float32),
            jax.ShapeDtypeStruct((NC, NROWS, DH), jnp.float32),
        ],
    )(ds, dd, feat)


def _dense_body(want_h, parts, si, so, w, a, out_ref, pool_ref):
    i = pl.program_id(0)
    agg = jnp.concatenate([parts[0], parts[1]], axis=-1) * si[...]
    out = jnp.dot(agg, w[...], preferred_element_type=jnp.float32)
    aa = a[0, 0]
    h = jnp.where(out >= 0.0, out, aa * out)
    if want_h:
        out_ref[...] = h
    else:
        hs = h * so[...]
        out_ref[0] = hs[:, :DH]
        out_ref[1] = hs[:, DH:]

    @pl.when(i == 0)
    def _():
        pool_ref[...] = jnp.zeros_like(pool_ref)

    pool_ref[...] += jnp.sum(h, axis=0, keepdims=True)


@functools.partial(jax.jit, static_argnums=0)
def _dense(want_h, parts, si, so, w, a):
    vec = pl.BlockSpec((_RB, 1), lambda i: (i, 0))
    if want_h:
        out_spec = pl.BlockSpec((_RB, D), lambda i: (i, 0))
        out_shape = jax.ShapeDtypeStruct((N, D), jnp.float32)
    else:
        out_spec = pl.BlockSpec((NC, _RB, DH), lambda i: (0, i, 0))
        out_shape = jax.ShapeDtypeStruct((NC, NROWS, DH), jnp.float32)
    return pl.pallas_call(
        functools.partial(_dense_body, want_h),
        grid=(N // _RB,),
        in_specs=[
            pl.BlockSpec((NC, _RB, DH), lambda i: (0, i, 0)),
            vec, vec,
            pl.BlockSpec((D, D), lambda i: (0, 0)),
            pl.BlockSpec(memory_space=pltpu.SMEM),
        ],
        out_specs=[out_spec, pl.BlockSpec((1, D), lambda i: (0, 0))],
        out_shape=[out_shape, jax.ShapeDtypeStruct((1, D), jnp.float32)],
    )(parts, si, so, w, a)


def kernel(feat, edge_index, W0, W1, a0, a1):
    ei_p = jnp.pad(edge_index, ((0, 0), (0, EPAD - E)),
                   constant_values=DUMP).reshape(2, NS, ECH, K)

    ones_cw = jnp.ones((K, CW), jnp.float32)
    zeros_cw = jnp.zeros((DRPT, CW), jnp.float32)
    zeros_kd = jnp.zeros((K, DH), jnp.float32)

    dcnt = _deg_kernel(ei_p, ones_cw, zeros_cw)
    s_out, s_in, fs2 = _prep(dcnt[0, :N, 0:1], dcnt[1, :N, 0:1], feat)

    a0_2d = a0.reshape(1, 1)
    a1_2d = a1.reshape(1, 1)

    parts1 = _edge_kernel(fs2, ei_p, zeros_kd)
    hs2, pool1 = _dense(False, parts1, s_in, s_out, W0, a0_2d)

    parts2 = _edge_kernel(hs2, ei_p, zeros_kd)
    h2, pool2 = _dense(True, parts2, s_in, s_out, W1, a1_2d)

    hg = jnp.concatenate([pool1, pool2], axis=-1)
    return (h2, hg)


# speedup vs baseline: 7.7597x; 1.0037x over previous
import functools

import jax
import jax.numpy as jnp
from jax import lax
from jax.experimental import pallas as pl
from jax.experimental.pallas import tpu as pltpu, tpu_sc as plsc

N = 10000
E = 320000
D = 128
DH = D // 2

NC = 2
NS = 16
NW = NC * NS

K = 128
EPT = E // NS
ECH = (EPT + K - 1) // K
EPAD = NS * ECH * K

RPT = 632
NROWS = RPT * NS
DUMP = N

CW = 8
DRPT = NROWS // NS

_mesh = plsc.VectorSubcoreMesh(core_axis_name="c", subcore_axis_name="s")
_sc_params = pltpu.CompilerParams(use_tc_tiling_on_sc=False)


@functools.partial(
    pl.kernel,
    out_type=jax.ShapeDtypeStruct((NC, NROWS, CW), jnp.float32),
    mesh=_mesh,
    compiler_params=_sc_params,
    scratch_types=[
        pltpu.VMEM((ECH, K), jnp.int32),
        pltpu.VMEM((K, CW), jnp.float32),
        pltpu.VMEM((DRPT, CW), jnp.float32),
        pltpu.VMEM_SHARED((NROWS, CW), jnp.float32),
        pltpu.SemaphoreType.DMA,
    ],
)
def _deg_kernel(ei_hbm, ones_hbm, zeros_hbm, out_hbm, idx_v, ones_v, z_v,
                table, sem):
    c = lax.axis_index("c")
    s = lax.axis_index("s")

    idx_load = pltpu.async_copy(ei_hbm.at[c, s], idx_v, sem)
    pltpu.sync_copy(ones_hbm, ones_v)
    pltpu.sync_copy(zeros_hbm, z_v)

    pltpu.sync_copy(z_v, table.at[pl.ds(s * DRPT, DRPT)])
    idx_load.wait()
    plsc.subcore_barrier()

    def body(ch, carry):
        pltpu.sync_copy(ones_v, table.at[idx_v.at[ch]], add=True)
        return carry

    lax.fori_loop(0, ECH, body, 0)
    plsc.subcore_barrier()

    pltpu.sync_copy(table.at[pl.ds(s * DRPT, DRPT)], z_v)
    pltpu.sync_copy(z_v, out_hbm.at[c, pl.ds(s * DRPT, DRPT)])


@functools.partial(
    pl.kernel,
    out_type=jax.ShapeDtypeStruct((NC, NROWS, DH), jnp.float32),
    mesh=_mesh,
    compiler_params=_sc_params,
    scratch_types=(
        [pltpu.VMEM((ECH, K), jnp.int32)] * 2
        + [pltpu.VMEM((K, DH), jnp.float32)] * 6
        + [pltpu.VMEM_SHARED((NROWS, DH), jnp.float32)]
        + [pltpu.SemaphoreType.DMA] * 13
    ),
)
def _edge_kernel(hs2_hbm, ei_hbm, zeros_hbm, out_hbm, src_v, dst_v, *rest):
    bufs = rest[0:6]
    acc = rest[6]
    gsem = rest[7:13]
    ssem = rest[13:19]
    isem = rest[19]
    c = lax.axis_index("c")
    s = lax.axis_index("s")
    page = hs2_hbm.at[c]

    src_load = pltpu.async_copy(ei_hbm.at[0, s], src_v, isem)
    dst_load = pltpu.async_copy(ei_hbm.at[1, s], dst_v, isem)
    pltpu.sync_copy(zeros_hbm, bufs[0])

    base = s * RPT
    for j in range(4):
        pltpu.sync_copy(bufs[0], acc.at[pl.ds(base + j * K, K)])
    pltpu.sync_copy(bufs[0].at[pl.ds(0, RPT - 4 * K)],
                    acc.at[pl.ds(base + 4 * K, RPT - 4 * K)])
    src_load.wait()
    dst_load.wait()
    plsc.subcore_barrier()

    def issue_g(ch, b):
        pltpu.async_copy(page.at[src_v.at[ch]], bufs[b], gsem[b])

    def issue_s(ch, b):
        pltpu.async_copy(bufs[b], acc.at[dst_v.at[ch]], ssem[b], add=True)

    def gwait(b):
        pltpu.make_async_copy(page.at[src_v.at[0]], bufs[b], gsem[b]).wait()

    def swait(b):
        pltpu.make_async_copy(bufs[b], acc.at[dst_v.at[0]], ssem[b]).wait()

    for ch in range(4):
        issue_g(ch, ch)

    gwait(0); issue_s(0, 0); issue_g(4, 4)
    gwait(1); issue_s(1, 1); issue_g(5, 5)
    gwait(2); issue_s(2, 2); swait(0); issue_g(6, 0)
    gwait(3); issue_s(3, 3); swait(1); issue_g(7, 1)
    gwait(4); issue_s(4, 4); swait(2); issue_g(8, 2)
    gwait(5); issue_s(5, 5); swait(3); issue_g(9, 3)

    def body(i, carry):
        ch = 6 * i
        for k in range(6):
            gwait(k)
            issue_s(ch + k, k)
            swait((k + 4) % 6)
            issue_g(ch + k + 4, (k + 4) % 6)
        return carry

    lax.fori_loop(1, 25, body, 0)

    for ch in range(150, 157):
        k = ch % 6
        gwait(k)
        issue_s(ch, k)
        swait((k + 4) % 6)
        if ch + 4 <= 156:
            issue_g(ch + 4, (k + 4) % 6)
    swait(5)
    swait(0)
    plsc.subcore_barrier()

    pltpu.sync_copy(acc.at[pl.ds(base, RPT)], out_hbm.at[c, pl.ds(base, RPT)])


_RB = 1000


def _prep_body(ds, dd, feat, so_ref, si_ref, fs_ref):
    so = lax.rsqrt(jnp.maximum(ds[...], 1.0))
    si = lax.rsqrt(jnp.maximum(dd[...], 1.0))
    so_ref[...] = so
    si_ref[...] = si
    fs_ref[0] = feat[:, :DH] * so
    fs_ref[1] = feat[:, DH:] * so


@jax.jit
def _prep(ds, dd, feat):
    vec = pl.BlockSpec((_RB, 1), lambda i: (i, 0))
    return pl.pallas_call(
        _prep_body,
        grid=(N // _RB,),
        in_specs=[vec, vec, pl.BlockSpec((_RB, D), lambda i: (i, 0))],
        out_specs=[vec, vec, pl.BlockSpec((NC, _RB, DH), lambda i: (0, i, 0))],
        out_shape=[
            jax.ShapeDtypeStruct((N, 1), jnp.float32),
            jax.ShapeDtypeStruct((N, 1), jnp.float32),
            jax.ShapeDtypeStruct((NC, NROWS, DH), jnp.float32),
        ],
    )(ds, dd, feat)


def _dense_body(want_h, parts, si, so, w, a, out_ref, pool_ref):
    i = pl.program_id(0)
    agg = jnp.concatenate([parts[0], parts[1]], axis=-1) * si[...]
    out = jnp.dot(agg, w[...], preferred_element_type=jnp.float32)
    aa = a[0, 0]
    h = jnp.where(out >= 0.0, out, aa * out)
    if want_h:
        out_ref[...] = h
    else:
        hs = h * so[...]
        out_ref[0] = hs[:, :DH]
        out_ref[1] = hs[:, DH:]

    @pl.when(i == 0)
    def _():
        pool_ref[...] = jnp.zeros_like(pool_ref)

    pool_ref[...] += jnp.sum(h, axis=0, keepdims=True)


@functools.partial(jax.jit, static_argnums=0)
def _dense(want_h, parts, si, so, w, a):
    vec = pl.BlockSpec((_RB, 1), lambda i: (i, 0))
    if want_h:
        out_spec = pl.BlockSpec((_RB, D), lambda i: (i, 0))
        out_shape = jax.ShapeDtypeStruct((N, D), jnp.float32)
    else:
        out_spec = pl.BlockSpec((NC, _RB, DH), lambda i: (0, i, 0))
        out_shape = jax.ShapeDtypeStruct((NC, NROWS, DH), jnp.float32)
    return pl.pallas_call(
        functools.partial(_dense_body, want_h),
        grid=(N // _RB,),
        in_specs=[
            pl.BlockSpec((NC, _RB, DH), lambda i: (0, i, 0)),
            vec, vec,
            pl.BlockSpec((D, D), lambda i: (0, 0)),
            pl.BlockSpec(memory_space=pltpu.SMEM),
        ],
        out_specs=[out_spec, pl.BlockSpec((1, D), lambda i: (0, 0))],
        out_shape=[out_shape, jax.ShapeDtypeStruct((1, D), jnp.float32)],
    )(parts, si, so, w, a)


def kernel(feat, edge_index, W0, W1, a0, a1):
    ei_p = jnp.pad(edge_index, ((0, 0), (0, EPAD - E)),
                   constant_values=DUMP).reshape(2, NS, ECH, K)

    ones_cw = jnp.ones((K, CW), jnp.float32)
    zeros_cw = jnp.zeros((DRPT, CW), jnp.float32)
    zeros_kd = jnp.zeros((K, DH), jnp.float32)

    dcnt = _deg_kernel(ei_p, ones_cw, zeros_cw)
    s_out, s_in, fs2 = _prep(dcnt[0, :N, 0:1], dcnt[1, :N, 0:1], feat)

    a0_2d = a0.reshape(1, 1)
    a1_2d = a1.reshape(1, 1)

    parts1 = _edge_kernel(fs2, ei_p, zeros_kd)
    hs2, pool1 = _dense(False, parts1, s_in, s_out, W0, a0_2d)

    parts2 = _edge_kernel(hs2, ei_p, zeros_kd)
    h2, pool2 = _dense(True, parts2, s_in, s_out, W1, a1_2d)

    hg = jnp.concatenate([pool1, pool2], axis=-1)
    return (h2, hg)


# speedup vs baseline: 7.8288x; 1.0089x over previous
import functools

import jax
import jax.numpy as jnp
from jax import lax
from jax.experimental import pallas as pl
from jax.experimental.pallas import tpu as pltpu, tpu_sc as plsc

N = 10000
E = 320000
D = 128
DH = D // 2

NC = 2
NS = 16
NW = NC * NS

K = 128
EPT = E // NS
ECH = (EPT + K - 1) // K
EPAD = NS * ECH * K

RPT = 632
NROWS = RPT * NS
DUMP = N

CW = 8
DRPT = NROWS // NS

_mesh = plsc.VectorSubcoreMesh(core_axis_name="c", subcore_axis_name="s")
_sc_params = pltpu.CompilerParams(use_tc_tiling_on_sc=False)


@functools.partial(
    pl.kernel,
    out_type=jax.ShapeDtypeStruct((NC, NROWS, CW), jnp.float32),
    mesh=_mesh,
    compiler_params=_sc_params,
    scratch_types=[
        pltpu.VMEM((ECH, K), jnp.int32),
        pltpu.VMEM((K, CW), jnp.float32),
        pltpu.VMEM((DRPT, CW), jnp.float32),
        pltpu.VMEM_SHARED((NROWS, CW), jnp.float32),
        pltpu.SemaphoreType.DMA,
    ],
)
def _deg_kernel(ei_hbm, ones_hbm, zeros_hbm, out_hbm, idx_v, ones_v, z_v,
                table, sem):
    c = lax.axis_index("c")
    s = lax.axis_index("s")

    idx_load = pltpu.async_copy(ei_hbm.at[c, s], idx_v, sem)
    pltpu.sync_copy(ones_hbm, ones_v)
    pltpu.sync_copy(zeros_hbm, z_v)

    pltpu.sync_copy(z_v, table.at[pl.ds(s * DRPT, DRPT)])
    idx_load.wait()
    plsc.subcore_barrier()

    def body(ch, carry):
        pltpu.sync_copy(ones_v, table.at[idx_v.at[ch]], add=True)
        return carry

    lax.fori_loop(0, ECH, body, 0)
    plsc.subcore_barrier()

    pltpu.sync_copy(table.at[pl.ds(s * DRPT, DRPT)], z_v)
    pltpu.sync_copy(z_v, out_hbm.at[c, pl.ds(s * DRPT, DRPT)])


@functools.partial(
    pl.kernel,
    out_type=jax.ShapeDtypeStruct((NC, NROWS, DH), jnp.float32),
    mesh=_mesh,
    compiler_params=_sc_params,
    scratch_types=(
        [pltpu.VMEM((ECH, K), jnp.int32)] * 2
        + [pltpu.VMEM((K, DH), jnp.float32)] * 6
        + [pltpu.VMEM_SHARED((NROWS, DH), jnp.float32)]
        + [pltpu.SemaphoreType.DMA] * 13
    ),
)
def _edge_kernel(hs2_hbm, ei_hbm, zeros_hbm, out_hbm, src_v, dst_v, *rest):
    bufs = rest[0:6]
    acc = rest[6]
    gsem = rest[7:13]
    ssem = rest[13:19]
    isem = rest[19]
    c = lax.axis_index("c")
    s = lax.axis_index("s")
    page = hs2_hbm.at[c]

    src_load = pltpu.async_copy(ei_hbm.at[0, s], src_v, isem)
    dst_load = pltpu.async_copy(ei_hbm.at[1, s], dst_v, isem)
    pltpu.sync_copy(zeros_hbm, bufs[0])

    base = s * RPT
    for j in range(4):
        pltpu.sync_copy(bufs[0], acc.at[pl.ds(base + j * K, K)])
    pltpu.sync_copy(bufs[0].at[pl.ds(0, RPT - 4 * K)],
                    acc.at[pl.ds(base + 4 * K, RPT - 4 * K)])
    src_load.wait()
    dst_load.wait()
    plsc.subcore_barrier()

    def issue_g(ch, b):
        pltpu.async_copy(page.at[src_v.at[ch]], bufs[b], gsem[b])

    def issue_s(ch, b):
        pltpu.async_copy(bufs[b], acc.at[dst_v.at[ch]], ssem[b], add=True)

    def gwait(b):
        pltpu.make_async_copy(page.at[src_v.at[0]], bufs[b], gsem[b]).wait()

    def swait(b):
        pltpu.make_async_copy(bufs[b], acc.at[dst_v.at[0]], ssem[b]).wait()

    for ch in range(4):
        issue_g(ch, ch)

    gwait(0); issue_s(0, 0); issue_g(4, 4)
    gwait(1); issue_s(1, 1); issue_g(5, 5)
    gwait(2); issue_s(2, 2); swait(0); issue_g(6, 0)
    gwait(3); issue_s(3, 3); swait(1); issue_g(7, 1)
    gwait(4); issue_s(4, 4); swait(2); issue_g(8, 2)
    gwait(5); issue_s(5, 5); swait(3); issue_g(9, 3)

    def body(i, carry):
        ch = 6 * i
        for k in range(6):
            gwait(k)
            issue_s(ch + k, k)
            swait((k + 4) % 6)
            issue_g(ch + k + 4, (k + 4) % 6)
        return carry

    lax.fori_loop(1, 25, body, 0)

    for ch in range(150, 157):
        k = ch % 6
        gwait(k)
        issue_s(ch, k)
        swait((k + 4) % 6)
        if ch + 4 <= 156:
            issue_g(ch + 4, (k + 4) % 6)
    swait(5)
    swait(0)
    plsc.subcore_barrier()

    pltpu.sync_copy(acc.at[pl.ds(base, RPT)], out_hbm.at[c, pl.ds(base, RPT)])


_RB = 2000


def _prep_body(ds, dd, feat, so_ref, si_ref, fs_ref):
    so = lax.rsqrt(jnp.maximum(ds[...], 1.0))
    si = lax.rsqrt(jnp.maximum(dd[...], 1.0))
    so_ref[...] = so
    si_ref[...] = si
    fs_ref[0] = feat[:, :DH] * so
    fs_ref[1] = feat[:, DH:] * so


@jax.jit
def _prep(ds, dd, feat):
    vec = pl.BlockSpec((_RB, 1), lambda i: (i, 0))
    return pl.pallas_call(
        _prep_body,
        grid=(N // _RB,),
        in_specs=[vec, vec, pl.BlockSpec((_RB, D), lambda i: (i, 0))],
        out_specs=[vec, vec, pl.BlockSpec((NC, _RB, DH), lambda i: (0, i, 0))],
        out_shape=[
            jax.ShapeDtypeStruct((N, 1), jnp.float32),
            jax.ShapeDtypeStruct((N, 1), jnp.float32),
            jax.ShapeDtypeStruct((NC, NROWS, DH), jnp.float32),
        ],
    )(ds, dd, feat)


def _dense_body(want_h, parts, si, so, w, a, out_ref, pool_ref):
    i = pl.program_id(0)
    agg = jnp.concatenate([parts[0], parts[1]], axis=-1) * si[...]
    out = jnp.dot(agg, w[...], preferred_element_type=jnp.float32)
    aa = a[0, 0]
    h = jnp.where(out >= 0.0, out, aa * out)
    if want_h:
        out_ref[...] = h
    else:
        hs = h * so[...]
        out_ref[0] = hs[:, :DH]
        out_ref[1] = hs[:, DH:]

    @pl.when(i == 0)
    def _():
        pool_ref[...] = jnp.zeros_like(pool_ref)

    pool_ref[...] += jnp.sum(h, axis=0, keepdims=True)


@functools.partial(jax.jit, static_argnums=0)
def _dense(want_h, parts, si, so, w, a):
    vec = pl.BlockSpec((_RB, 1), lambda i: (i, 0))
    if want_h:
        out_spec = pl.BlockSpec((_RB, D), lambda i: (i, 0))
        out_shape = jax.ShapeDtypeStruct((N, D), jnp.float32)
    else:
        out_spec = pl.BlockSpec((NC, _RB, DH), lambda i: (0, i, 0))
        out_shape = jax.ShapeDtypeStruct((NC, NROWS, DH), jnp.float32)
    return pl.pallas_call(
        functools.partial(_dense_body, want_h),
        grid=(N // _RB,),
        in_specs=[
            pl.BlockSpec((NC, _RB, DH), lambda i: (0, i, 0)),
            vec, vec,
            pl.BlockSpec((D, D), lambda i: (0, 0)),
            pl.BlockSpec(memory_space=pltpu.SMEM),
        ],
        out_specs=[out_spec, pl.BlockSpec((1, D), lambda i: (0, 0))],
        out_shape=[out_shape, jax.ShapeDtypeStruct((1, D), jnp.float32)],
    )(parts, si, so, w, a)


def kernel(feat, edge_index, W0, W1, a0, a1):
    ei_p = jnp.pad(edge_index, ((0, 0), (0, EPAD - E)),
                   constant_values=DUMP).reshape(2, NS, ECH, K)
    ei_p = jax.lax.optimization_barrier(ei_p)

    ones_cw = jnp.ones((K, CW), jnp.float32)
    zeros_cw = jnp.zeros((DRPT, CW), jnp.float32)
    zeros_kd = jnp.zeros((K, DH), jnp.float32)

    dcnt = _deg_kernel(ei_p, ones_cw, zeros_cw)
    s_out, s_in, fs2 = _prep(dcnt[0, :N, 0:1], dcnt[1, :N, 0:1], feat)

    a0_2d = a0.reshape(1, 1)
    a1_2d = a1.reshape(1, 1)

    parts1 = _edge_kernel(fs2, ei_p, zeros_kd)
    hs2, pool1 = _dense(False, parts1, s_in, s_out, W0, a0_2d)

    parts2 = _edge_kernel(hs2, ei_p, zeros_kd)
    h2, pool2 = _dense(True, parts2, s_in, s_out, W1, a1_2d)

    hg = jnp.concatenate([pool1, pool2], axis=-1)
    return (h2, hg)


# speedup vs baseline: 7.8388x; 1.0013x over previous
import functools

import jax
import jax.numpy as jnp
from jax import lax
from jax.experimental import pallas as pl
from jax.experimental.pallas import tpu as pltpu, tpu_sc as plsc

N = 10000
E = 320000
D = 128
DH = D // 2

NC = 2
NS = 16
NW = NC * NS

K = 128
EPT = E // NS
ECH = (EPT + K - 1) // K
EPAD = NS * ECH * K

RPT = 632
NROWS = RPT * NS
DUMP = N

CW = 8
DRPT = NROWS // NS

_mesh = plsc.VectorSubcoreMesh(core_axis_name="c", subcore_axis_name="s")
_sc_params = pltpu.CompilerParams(use_tc_tiling_on_sc=False)


@functools.partial(
    pl.kernel,
    out_type=jax.ShapeDtypeStruct((NC, NROWS, CW), jnp.float32),
    mesh=_mesh,
    compiler_params=_sc_params,
    scratch_types=[
        pltpu.VMEM((ECH, K), jnp.int32),
        pltpu.VMEM((K, CW), jnp.float32),
        pltpu.VMEM((DRPT, CW), jnp.float32),
        pltpu.VMEM_SHARED((NROWS, CW), jnp.float32),
        pltpu.SemaphoreType.DMA,
    ],
)
def _deg_kernel(ei_hbm, ones_hbm, zeros_hbm, out_hbm, idx_v, ones_v, z_v,
                table, sem):
    c = lax.axis_index("c")
    s = lax.axis_index("s")

    idx_load = pltpu.async_copy(ei_hbm.at[c, s], idx_v, sem)
    pltpu.sync_copy(ones_hbm, ones_v)
    pltpu.sync_copy(zeros_hbm, z_v)

    pltpu.sync_copy(z_v, table.at[pl.ds(s * DRPT, DRPT)])
    idx_load.wait()
    plsc.subcore_barrier()

    def body(ch, carry):
        pltpu.sync_copy(ones_v, table.at[idx_v.at[ch]], add=True)
        return carry

    lax.fori_loop(0, ECH, body, 0)
    plsc.subcore_barrier()

    pltpu.sync_copy(table.at[pl.ds(s * DRPT, DRPT)], z_v)
    pltpu.sync_copy(z_v, out_hbm.at[c, pl.ds(s * DRPT, DRPT)])


@functools.partial(
    pl.kernel,
    out_type=jax.ShapeDtypeStruct((NC, NROWS, DH), jnp.float32),
    mesh=_mesh,
    compiler_params=_sc_params,
    scratch_types=(
        [pltpu.VMEM((ECH, K), jnp.int32)] * 2
        + [pltpu.VMEM((K, DH), jnp.float32)] * 6
        + [pltpu.VMEM_SHARED((NROWS, DH), jnp.float32)]
        + [pltpu.SemaphoreType.DMA] * 19
    ),
)
def _edge_kernel(hs2_hbm, ei_hbm, zeros_hbm, out_hbm, src_v, dst_v, *rest):
    bufs = rest[0:6]
    acc = rest[6]
    gsa = rest[7:13]
    gsb = rest[13:19]
    ssem = rest[19:25]
    isem = rest[25]
    c = lax.axis_index("c")
    s = lax.axis_index("s")
    page = hs2_hbm.at[c]

    src_load = pltpu.async_copy(ei_hbm.at[0, s], src_v, isem)
    dst_load = pltpu.async_copy(ei_hbm.at[1, s], dst_v, isem)
    pltpu.sync_copy(zeros_hbm, bufs[0])

    base = s * RPT
    for j in range(4):
        pltpu.sync_copy(bufs[0], acc.at[pl.ds(base + j * K, K)])
    pltpu.sync_copy(bufs[0].at[pl.ds(0, RPT - 4 * K)],
                    acc.at[pl.ds(base + 4 * K, RPT - 4 * K)])
    src_load.wait()
    dst_load.wait()
    plsc.subcore_barrier()

    HK = K // 2

    def issue_g(ch, b):
        pltpu.async_copy(page.at[src_v.at[ch, pl.ds(0, HK)]],
                         bufs[b].at[pl.ds(0, HK)], gsa[b])
        pltpu.async_copy(page.at[src_v.at[ch, pl.ds(HK, HK)]],
                         bufs[b].at[pl.ds(HK, HK)], gsb[b])

    def issue_s(ch, b):
        pltpu.async_copy(bufs[b], acc.at[dst_v.at[ch]], ssem[b], add=True)

    def gwait(b):
        pltpu.make_async_copy(page.at[src_v.at[0, pl.ds(0, HK)]],
                              bufs[b].at[pl.ds(0, HK)], gsa[b]).wait()
        pltpu.make_async_copy(page.at[src_v.at[0, pl.ds(0, HK)]],
                              bufs[b].at[pl.ds(HK, HK)], gsb[b]).wait()

    def swait(b):
        pltpu.make_async_copy(bufs[b], acc.at[dst_v.at[0]], ssem[b]).wait()

    for ch in range(4):
        issue_g(ch, ch)

    gwait(0); issue_s(0, 0); issue_g(4, 4)
    gwait(1); issue_s(1, 1); issue_g(5, 5)
    gwait(2); issue_s(2, 2); swait(0); issue_g(6, 0)
    gwait(3); issue_s(3, 3); swait(1); issue_g(7, 1)
    gwait(4); issue_s(4, 4); swait(2); issue_g(8, 2)
    gwait(5); issue_s(5, 5); swait(3); issue_g(9, 3)

    def body(i, carry):
        ch = 6 * i
        for k in range(6):
            gwait(k)
            issue_s(ch + k, k)
            swait((k + 4) % 6)
            issue_g(ch + k + 4, (k + 4) % 6)
        return carry

    lax.fori_loop(1, 25, body, 0)

    for ch in range(150, 157):
        k = ch % 6
        gwait(k)
        issue_s(ch, k)
        swait((k + 4) % 6)
        if ch + 4 <= 156:
            issue_g(ch + 4, (k + 4) % 6)
    swait(5)
    swait(0)
    plsc.subcore_barrier()

    pltpu.sync_copy(acc.at[pl.ds(base, RPT)], out_hbm.at[c, pl.ds(base, RPT)])


_RB = 2000


def _prep_body(ds, dd, feat, so_ref, si_ref, fs_ref):
    so = lax.rsqrt(jnp.maximum(ds[...], 1.0))
    si = lax.rsqrt(jnp.maximum(dd[...], 1.0))
    so_ref[...] = so
    si_ref[...] = si
    fs_ref[0] = feat[:, :DH] * so
    fs_ref[1] = feat[:, DH:] * so


@jax.jit
def _prep(ds, dd, feat):
    vec = pl.BlockSpec((_RB, 1), lambda i: (i, 0))
    return pl.pallas_call(
        _prep_body,
        grid=(N // _RB,),
        in_specs=[vec, vec, pl.BlockSpec((_RB, D), lambda i: (i, 0))],
        out_specs=[vec, vec, pl.BlockSpec((NC, _RB, DH), lambda i: (0, i, 0))],
        out_shape=[
            jax.ShapeDtypeStruct((N, 1), jnp.float32),
            jax.ShapeDtypeStruct((N, 1), jnp.float32),
            jax.ShapeDtypeStruct((NC, NROWS, DH), jnp.float32),
        ],
    )(ds, dd, feat)


def _dense_body(want_h, parts, si, so, w, a, out_ref, pool_ref):
    i = pl.program_id(0)
    agg = jnp.concatenate([parts[0], parts[1]], axis=-1) * si[...]
    out = jnp.dot(agg, w[...], preferred_element_type=jnp.float32)
    aa = a[0, 0]
    h = jnp.where(out >= 0.0, out, aa * out)
    if want_h:
        out_ref[...] = h
    else:
        hs = h * so[...]
        out_ref[0] = hs[:, :DH]
        out_ref[1] = hs[:, DH:]

    @pl.when(i == 0)
    def _():
        pool_ref[...] = jnp.zeros_like(pool_ref)

    pool_ref[...] += jnp.sum(h, axis=0, keepdims=True)


@functools.partial(jax.jit, static_argnums=0)
def _dense(want_h, parts, si, so, w, a):
    vec = pl.BlockSpec((_RB, 1), lambda i: (i, 0))
    if want_h:
        out_spec = pl.BlockSpec((_RB, D), lambda i: (i, 0))
        out_shape = jax.ShapeDtypeStruct((N, D), jnp.float32)
    else:
        out_spec = pl.BlockSpec((NC, _RB, DH), lambda i: (0, i, 0))
        out_shape = jax.ShapeDtypeStruct((NC, NROWS, DH), jnp.float32)
    return pl.pallas_call(
        functools.partial(_dense_body, want_h),
        grid=(N // _RB,),
        in_specs=[
            pl.BlockSpec((NC, _RB, DH), lambda i: (0, i, 0)),
            vec, vec,
            pl.BlockSpec((D, D), lambda i: (0, 0)),
            pl.BlockSpec(memory_space=pltpu.SMEM),
        ],
        out_specs=[out_spec, pl.BlockSpec((1, D), lambda i: (0, 0))],
        out_shape=[out_shape, jax.ShapeDtypeStruct((1, D), jnp.float32)],
    )(parts, si, so, w, a)


def kernel(feat, edge_index, W0, W1, a0, a1):
    ei_p = jnp.pad(edge_index, ((0, 0), (0, EPAD - E)),
                   constant_values=DUMP).reshape(2, NS, ECH, K)
    ei_p = jax.lax.optimization_barrier(ei_p)

    ones_cw = jnp.ones((K, CW), jnp.float32)
    zeros_cw = jnp.zeros((DRPT, CW), jnp.float32)
    zeros_kd = jnp.zeros((K, DH), jnp.float32)

    dcnt = _deg_kernel(ei_p, ones_cw, zeros_cw)
    s_out, s_in, fs2 = _prep(dcnt[0, :N, 0:1], dcnt[1, :N, 0:1], feat)

    a0_2d = a0.reshape(1, 1)
    a1_2d = a1.reshape(1, 1)

    parts1 = _edge_kernel(fs2, ei_p, zeros_kd)
    hs2, pool1 = _dense(False, parts1, s_in, s_out, W0, a0_2d)

    parts2 = _edge_kernel(hs2, ei_p, zeros_kd)
    h2, pool2 = _dense(True, parts2, s_in, s_out, W1, a1_2d)

    hg = jnp.concatenate([pool1, pool2], axis=-1)
    return (h2, hg)


# speedup vs baseline: 8.0332x; 1.0248x over previous
import functools

import jax
import jax.numpy as jnp
from jax import lax
from jax.experimental import pallas as pl
from jax.experimental.pallas import tpu as pltpu, tpu_sc as plsc

N = 10000
E = 320000
D = 128
DH = D // 2

NC = 2
NS = 16
NW = NC * NS

K = 128
EPT = E // NS
ECH = (EPT + K - 1) // K
EPAD = NS * ECH * K

RPT = 632
NROWS = RPT * NS
DUMP = N

CW = 8
DRPT = NROWS // NS

_mesh = plsc.VectorSubcoreMesh(core_axis_name="c", subcore_axis_name="s")
_sc_params = pltpu.CompilerParams(use_tc_tiling_on_sc=False)


@functools.partial(
    pl.kernel,
    out_type=jax.ShapeDtypeStruct((NC, NROWS, CW), jnp.float32),
    mesh=_mesh,
    compiler_params=_sc_params,
    scratch_types=[
        pltpu.VMEM((ECH, K), jnp.int32),
        pltpu.VMEM((K, CW), jnp.float32),
        pltpu.VMEM((DRPT, CW), jnp.float32),
        pltpu.VMEM_SHARED((NROWS, CW), jnp.float32),
        pltpu.SemaphoreType.DMA,
    ],
)
def _deg_kernel(ei_hbm, ones_hbm, zeros_hbm, out_hbm, idx_v, ones_v, z_v,
                table, sem):
    c = lax.axis_index("c")
    s = lax.axis_index("s")

    idx_load = pltpu.async_copy(ei_hbm.at[c, s], idx_v, sem)
    pltpu.sync_copy(ones_hbm, ones_v)
    pltpu.sync_copy(zeros_hbm, z_v)

    pltpu.sync_copy(z_v, table.at[pl.ds(s * DRPT, DRPT)])
    idx_load.wait()
    plsc.subcore_barrier()

    def body(ch, carry):
        pltpu.sync_copy(ones_v, table.at[idx_v.at[ch]], add=True)
        return carry

    lax.fori_loop(0, ECH, body, 0)
    plsc.subcore_barrier()

    pltpu.sync_copy(table.at[pl.ds(s * DRPT, DRPT)], z_v)
    pltpu.sync_copy(z_v, out_hbm.at[c, pl.ds(s * DRPT, DRPT)])


@functools.partial(
    pl.kernel,
    out_type=jax.ShapeDtypeStruct((NC, NROWS, DH), jnp.float32),
    mesh=_mesh,
    compiler_params=_sc_params,
    scratch_types=(
        [pltpu.VMEM((ECH, K), jnp.int32)] * 2
        + [pltpu.VMEM((K, DH), jnp.float32)] * 6
        + [pltpu.VMEM_SHARED((NROWS, DH), jnp.float32)]
        + [pltpu.SemaphoreType.DMA] * 19
    ),
)
def _edge_kernel(hs2_hbm, ei_hbm, zeros_hbm, out_hbm, src_v, dst_v, *rest):
    bufs = rest[0:6]
    acc = rest[6]
    gsa = rest[7:13]
    gsb = rest[13:19]
    ssem = rest[19:25]
    isem = rest[25]
    c = lax.axis_index("c")
    s = lax.axis_index("s")
    page = hs2_hbm.at[c]

    src_load = pltpu.async_copy(ei_hbm.at[0, s], src_v, isem)
    dst_load = pltpu.async_copy(ei_hbm.at[1, s], dst_v, isem)
    pltpu.sync_copy(zeros_hbm, bufs[0])

    base = s * RPT
    for j in range(4):
        pltpu.sync_copy(bufs[0], acc.at[pl.ds(base + j * K, K)])
    pltpu.sync_copy(bufs[0].at[pl.ds(0, RPT - 4 * K)],
                    acc.at[pl.ds(base + 4 * K, RPT - 4 * K)])
    src_load.wait()
    dst_load.wait()
    plsc.subcore_barrier()

    HK = K // 2

    def issue_g(ch, b):
        pltpu.async_copy(page.at[src_v.at[ch, pl.ds(0, HK)]],
                         bufs[b].at[pl.ds(0, HK)], gsa[b])
        pltpu.async_copy(page.at[src_v.at[ch, pl.ds(HK, HK)]],
                         bufs[b].at[pl.ds(HK, HK)], gsb[b])

    def issue_s(ch, b):
        pltpu.async_copy(bufs[b], acc.at[dst_v.at[ch]], ssem[b], add=True)

    def gwait(b):
        pltpu.make_async_copy(page.at[src_v.at[0, pl.ds(0, HK)]],
                              bufs[b].at[pl.ds(0, HK)], gsa[b]).wait()
        pltpu.make_async_copy(page.at[src_v.at[0, pl.ds(0, HK)]],
                              bufs[b].at[pl.ds(HK, HK)], gsb[b]).wait()

    def swait(b):
        pltpu.make_async_copy(bufs[b], acc.at[dst_v.at[0]], ssem[b]).wait()

    for ch in range(4):
        issue_g(ch, ch)

    gwait(0); issue_s(0, 0); issue_g(4, 4)
    gwait(1); issue_s(1, 1); issue_g(5, 5)
    gwait(2); issue_s(2, 2); swait(0); issue_g(6, 0)
    gwait(3); issue_s(3, 3); swait(1); issue_g(7, 1)
    gwait(4); issue_s(4, 4); swait(2); issue_g(8, 2)
    gwait(5); issue_s(5, 5); swait(3); issue_g(9, 3)

    def body(i, carry):
        ch = 6 * i
        for k in range(6):
            gwait(k)
            issue_s(ch + k, k)
            swait((k + 4) % 6)
            issue_g(ch + k + 4, (k + 4) % 6)
        return carry

    lax.fori_loop(1, 25, body, 0)

    for ch in range(150, 157):
        k = ch % 6
        gwait(k)
        issue_s(ch, k)
        swait((k + 4) % 6)
        if ch + 4 <= 156:
            issue_g(ch + 4, (k + 4) % 6)
    swait(5)
    swait(0)
    plsc.subcore_barrier()

    pltpu.sync_copy(acc.at[pl.ds(base, RPT)], out_hbm.at[c, pl.ds(base, RPT)])


_RB = 2000


def _prep_body(ds, dd, feat, so_ref, si_ref, fs_ref):
    so = lax.rsqrt(jnp.maximum(ds[0, :, 0:1], 1.0))
    si = lax.rsqrt(jnp.maximum(dd[0, :, 0:1], 1.0))
    so_ref[...] = so
    si_ref[...] = si
    fs_ref[0] = feat[:, :DH] * so
    fs_ref[1] = feat[:, DH:] * so


@jax.jit
def _prep(dcnt, feat):
    vec = pl.BlockSpec((_RB, 1), lambda i: (i, 0))
    cs = pl.BlockSpec((1, _RB, CW), lambda i: (0, i, 0))
    cd = pl.BlockSpec((1, _RB, CW), lambda i: (1, i, 0))
    return pl.pallas_call(
        _prep_body,
        grid=(N // _RB,),
        in_specs=[cs, cd, pl.BlockSpec((_RB, D), lambda i: (i, 0))],
        out_specs=[vec, vec, pl.BlockSpec((NC, _RB, DH), lambda i: (0, i, 0))],
        out_shape=[
            jax.ShapeDtypeStruct((N, 1), jnp.float32),
            jax.ShapeDtypeStruct((N, 1), jnp.float32),
            jax.ShapeDtypeStruct((NC, NROWS, DH), jnp.float32),
        ],
    )(dcnt, dcnt, feat)


def _dense_body(want_h, parts, si, so, w, a, out_ref, pool_ref):
    i = pl.program_id(0)
    agg = jnp.concatenate([parts[0], parts[1]], axis=-1) * si[...]
    out = jnp.dot(agg, w[...], preferred_element_type=jnp.float32)
    aa = a[0, 0]
    h = jnp.where(out >= 0.0, out, aa * out)
    if want_h:
        out_ref[...] = h
    else:
        hs = h * so[...]
        out_ref[0] = hs[:, :DH]
        out_ref[1] = hs[:, DH:]

    @pl.when(i == 0)
    def _():
        pool_ref[...] = jnp.zeros_like(pool_ref)

    pool_ref[...] += jnp.sum(h, axis=0, keepdims=True)


@functools.partial(jax.jit, static_argnums=0)
def _dense(want_h, parts, si, so, w, a):
    vec = pl.BlockSpec((_RB, 1), lambda i: (i, 0))
    if want_h:
        out_spec = pl.BlockSpec((_RB, D), lambda i: (i, 0))
        out_shape = jax.ShapeDtypeStruct((N, D), jnp.float32)
    else:
        out_spec = pl.BlockSpec((NC, _RB, DH), lambda i: (0, i, 0))
        out_shape = jax.ShapeDtypeStruct((NC, NROWS, DH), jnp.float32)
    return pl.pallas_call(
        functools.partial(_dense_body, want_h),
        grid=(N // _RB,),
        in_specs=[
            pl.BlockSpec((NC, _RB, DH), lambda i: (0, i, 0)),
            vec, vec,
            pl.BlockSpec((D, D), lambda i: (0, 0)),
            pl.BlockSpec(memory_space=pltpu.SMEM),
        ],
        out_specs=[out_spec, pl.BlockSpec((1, D), lambda i: (0, 0))],
        out_shape=[out_shape, jax.ShapeDtypeStruct((1, D), jnp.float32)],
    )(parts, si, so, w, a)


def kernel(feat, edge_index, W0, W1, a0, a1):
    ei_p = jnp.pad(edge_index, ((0, 0), (0, EPAD - E)),
                   constant_values=DUMP).reshape(2, NS, ECH, K)
    ei_p = jax.lax.optimization_barrier(ei_p)

    ones_cw = jnp.ones((K, CW), jnp.float32)
    zeros_cw = jnp.zeros((DRPT, CW), jnp.float32)
    zeros_kd = jnp.zeros((K, DH), jnp.float32)

    dcnt = _deg_kernel(ei_p, ones_cw, zeros_cw)
    s_out, s_in, fs2 = _prep(dcnt, feat)

    a0_2d = a0.reshape(1, 1)
    a1_2d = a1.reshape(1, 1)

    parts1 = _edge_kernel(fs2, ei_p, zeros_kd)
    hs2, pool1 = _dense(False, parts1, s_in, s_out, W0, a0_2d)

    parts2 = _edge_kernel(hs2, ei_p, zeros_kd)
    h2, pool2 = _dense(True, parts2, s_in, s_out, W1, a1_2d)

    hg = jnp.concatenate([pool1, pool2], axis=-1)
    return (h2, hg)


# speedup vs baseline: 8.1225x; 1.0111x over previous
import functools

import jax
import jax.numpy as jnp
from jax import lax
from jax.experimental import pallas as pl
from jax.experimental.pallas import tpu as pltpu, tpu_sc as plsc

N = 10000
E = 320000
D = 128
DH = D // 2

NC = 2
NS = 16
NW = NC * NS

K = 128
EPT = E // NS
ECH = (EPT + K - 1) // K
EPAD = NS * ECH * K

RPT = 632
NROWS = RPT * NS
DUMP = N

CW = 8
DRPT = NROWS // NS

_mesh = plsc.VectorSubcoreMesh(core_axis_name="c", subcore_axis_name="s")
_sc_params = pltpu.CompilerParams(use_tc_tiling_on_sc=False)


@functools.partial(
    pl.kernel,
    out_type=jax.ShapeDtypeStruct((NC, NROWS, CW), jnp.float32),
    mesh=_mesh,
    compiler_params=_sc_params,
    scratch_types=[
        pltpu.VMEM((ECH, K), jnp.int32),
        pltpu.VMEM((K, CW), jnp.float32),
        pltpu.VMEM((DRPT, CW), jnp.float32),
        pltpu.VMEM_SHARED((NROWS, CW), jnp.float32),
        pltpu.SemaphoreType.DMA,
    ],
)
def _deg_kernel(ei_hbm, ones_hbm, zeros_hbm, out_hbm, idx_v, ones_v, z_v,
                table, sem):
    c = lax.axis_index("c")
    s = lax.axis_index("s")

    idx_load = pltpu.async_copy(ei_hbm.at[c, s], idx_v, sem)
    pltpu.sync_copy(ones_hbm, ones_v)
    pltpu.sync_copy(zeros_hbm, z_v)

    pltpu.sync_copy(z_v, table.at[pl.ds(s * DRPT, DRPT)])
    idx_load.wait()
    plsc.subcore_barrier()

    def body(ch, carry):
        pltpu.sync_copy(ones_v, table.at[idx_v.at[ch]], add=True)
        return carry

    lax.fori_loop(0, ECH, body, 0)
    plsc.subcore_barrier()

    pltpu.sync_copy(table.at[pl.ds(s * DRPT, DRPT)], z_v)
    pltpu.sync_copy(z_v, out_hbm.at[c, pl.ds(s * DRPT, DRPT)])


@functools.partial(
    pl.kernel,
    out_type=jax.ShapeDtypeStruct((NROWS, D), jnp.float32),
    mesh=_mesh,
    compiler_params=_sc_params,
    scratch_types=(
        [pltpu.VMEM((ECH, K), jnp.int32)] * 2
        + [pltpu.VMEM((K, DH), jnp.float32)] * 5
        + [pltpu.VMEM_SHARED((NROWS, DH), jnp.float32)]
        + [pltpu.SemaphoreType.DMA] * 11
    ),
)
def _edge_kernel(hsv_hbm, ei_hbm, zeros_hbm, out_hbm, src_v, dst_v, *rest):
    bufs = rest[0:5]
    acc = rest[5]
    gsem = rest[6:11]
    ssem = rest[11:16]
    isem = rest[16]
    c = lax.axis_index("c")
    s = lax.axis_index("s")
    col = c * DH

    src_load = pltpu.async_copy(ei_hbm.at[0, s], src_v, isem)
    dst_load = pltpu.async_copy(ei_hbm.at[1, s], dst_v, isem)
    pltpu.sync_copy(zeros_hbm, bufs[0])

    base = s * RPT
    for j in range(4):
        pltpu.sync_copy(bufs[0], acc.at[pl.ds(base + j * K, K)])
    pltpu.sync_copy(bufs[0].at[pl.ds(0, RPT - 4 * K)],
                    acc.at[pl.ds(base + 4 * K, RPT - 4 * K)])
    src_load.wait()
    dst_load.wait()

    def xform(i, carry):
        r = i // (K // 16)
        jj = (i % (K // 16)) * 16
        v = src_v[r, pl.ds(jj, 16)]
        src_v[r, pl.ds(jj, 16)] = v * 2 + c
        return carry

    lax.fori_loop(0, ECH * (K // 16), xform, 0)
    plsc.subcore_barrier()

    def issue_g(ch, b):
        pltpu.async_copy(hsv_hbm.at[src_v.at[ch]], bufs[b], gsem[b])

    def issue_s(ch, b):
        pltpu.async_copy(bufs[b], acc.at[dst_v.at[ch]], ssem[b], add=True)

    def gwait(b):
        pltpu.make_async_copy(hsv_hbm.at[src_v.at[0]], bufs[b], gsem[b]).wait()

    def swait(b):
        pltpu.make_async_copy(bufs[b], acc.at[dst_v.at[0]], ssem[b]).wait()

    for ch in range(4):
        issue_g(ch, ch)

    gwait(0); issue_s(0, 0); issue_g(4, 4)
    gwait(1); issue_s(1, 1); swait(0); issue_g(5, 0)
    gwait(2); issue_s(2, 2); swait(1); issue_g(6, 1)
    gwait(3); issue_s(3, 3); swait(2); issue_g(7, 2)
    gwait(4); issue_s(4, 4); swait(3); issue_g(8, 3)

    def body(i, carry):
        ch = 5 * i
        gwait(0); issue_s(ch, 0); swait(4); issue_g(ch + 4, 4)
        gwait(1); issue_s(ch + 1, 1); swait(0); issue_g(ch + 5, 0)
        gwait(2); issue_s(ch + 2, 2); swait(1); issue_g(ch + 6, 1)
        gwait(3); issue_s(ch + 3, 3); swait(2); issue_g(ch + 7, 2)
        gwait(4); issue_s(ch + 4, 4); swait(3); issue_g(ch + 8, 3)
        return carry

    lax.fori_loop(1, 30, body, 0)

    gwait(0); issue_s(150, 0); swait(4); issue_g(154, 4)
    gwait(1); issue_s(151, 1); swait(0); issue_g(155, 0)
    gwait(2); issue_s(152, 2); swait(1); issue_g(156, 1)
    gwait(3); issue_s(153, 3); swait(2)
    gwait(4); issue_s(154, 4); swait(3)
    gwait(0); issue_s(155, 0); swait(4)
    gwait(1); issue_s(156, 1); swait(0)
    swait(1)
    plsc.subcore_barrier()

    pltpu.sync_copy(acc.at[pl.ds(base, RPT)],
                    out_hbm.at[pl.ds(base, RPT), pl.ds(col, DH)])


_RB = 2000


def _prep_body(ds, dd, feat, so_ref, si_ref, fs_ref):
    so = lax.rsqrt(jnp.maximum(ds[0, :, 0:1], 1.0))
    si = lax.rsqrt(jnp.maximum(dd[0, :, 0:1], 1.0))
    so_ref[...] = so
    si_ref[...] = si
    fs_ref[...] = feat[...] * so


@jax.jit
def _prep(dcnt, feat):
    vec = pl.BlockSpec((_RB, 1), lambda i: (i, 0))
    cs = pl.BlockSpec((1, _RB, CW), lambda i: (0, i, 0))
    cd = pl.BlockSpec((1, _RB, CW), lambda i: (1, i, 0))
    return pl.pallas_call(
        _prep_body,
        grid=(N // _RB,),
        in_specs=[cs, cd, pl.BlockSpec((_RB, D), lambda i: (i, 0))],
        out_specs=[vec, vec, pl.BlockSpec((_RB, D), lambda i: (i, 0))],
        out_shape=[
            jax.ShapeDtypeStruct((N, 1), jnp.float32),
            jax.ShapeDtypeStruct((N, 1), jnp.float32),
            jax.ShapeDtypeStruct((NROWS, D), jnp.float32),
        ],
    )(dcnt, dcnt, feat)


def _dense_body(want_h, parts, si, so, w, a, out_ref, pool_ref):
    i = pl.program_id(0)
    agg = parts[...] * si[...]
    out = jnp.dot(agg, w[...], preferred_element_type=jnp.float32)
    aa = a[0, 0]
    h = jnp.where(out >= 0.0, out, aa * out)
    if want_h:
        out_ref[...] = h
    else:
        out_ref[...] = h * so[...]

    @pl.when(i == 0)
    def _():
        pool_ref[...] = jnp.zeros_like(pool_ref)

    pool_ref[...] += jnp.sum(h, axis=0, keepdims=True)


@functools.partial(jax.jit, static_argnums=0)
def _dense(want_h, parts, si, so, w, a):
    vec = pl.BlockSpec((_RB, 1), lambda i: (i, 0))
    if want_h:
        out_spec = pl.BlockSpec((_RB, D), lambda i: (i, 0))
        out_shape = jax.ShapeDtypeStruct((N, D), jnp.float32)
    else:
        out_spec = pl.BlockSpec((_RB, D), lambda i: (i, 0))
        out_shape = jax.ShapeDtypeStruct((NROWS, D), jnp.float32)
    return pl.pallas_call(
        functools.partial(_dense_body, want_h),
        grid=(N // _RB,),
        in_specs=[
            pl.BlockSpec((_RB, D), lambda i: (i, 0)),
            vec, vec,
            pl.BlockSpec((D, D), lambda i: (0, 0)),
            pl.BlockSpec(memory_space=pltpu.SMEM),
        ],
        out_specs=[out_spec, pl.BlockSpec((1, D), lambda i: (0, 0))],
        out_shape=[out_shape, jax.ShapeDtypeStruct((1, D), jnp.float32)],
    )(parts, si, so, w, a)


def kernel(feat, edge_index, W0, W1, a0, a1):
    ei_p = jnp.pad(edge_index, ((0, 0), (0, EPAD - E)),
                   constant_values=DUMP).reshape(2, NS, ECH, K)
    ei_p = jax.lax.optimization_barrier(ei_p)

    ones_cw = jnp.ones((K, CW), jnp.float32)
    zeros_cw = jnp.zeros((DRPT, CW), jnp.float32)
    zeros_kd = jnp.zeros((K, DH), jnp.float32)

    dcnt = _deg_kernel(ei_p, ones_cw, zeros_cw)
    s_out, s_in, fs2 = _prep(dcnt, feat)

    a0_2d = a0.reshape(1, 1)
    a1_2d = a1.reshape(1, 1)

    parts1 = _edge_kernel(fs2.reshape(2 * NROWS, DH), ei_p, zeros_kd)
    hs2, pool1 = _dense(False, parts1, s_in, s_out, W0, a0_2d)

    parts2 = _edge_kernel(hs2.reshape(2 * NROWS, DH), ei_p, zeros_kd)
    h2, pool2 = _dense(True, parts2, s_in, s_out, W1, a1_2d)

    hg = jnp.concatenate([pool1, pool2], axis=-1)
    return (h2, hg)


# speedup vs baseline: 8.3911x; 1.0331x over previous
import functools

import jax
import jax.numpy as jnp
from jax import lax
from jax.experimental import pallas as pl
from jax.experimental.pallas import tpu as pltpu, tpu_sc as plsc

N = 10000
E = 320000
D = 128
DH = D // 2

NC = 2
NS = 16
NW = NC * NS

K = 128
EPT = E // NS
ECH = (EPT + K - 1) // K
EPAD = NS * ECH * K

RPT = 632
NROWS = RPT * NS
DUMP = N

CW = 8
DRPT = NROWS // NS

_mesh = plsc.VectorSubcoreMesh(core_axis_name="c", subcore_axis_name="s")
_sc_params = pltpu.CompilerParams(use_tc_tiling_on_sc=False)


@functools.partial(
    pl.kernel,
    out_type=jax.ShapeDtypeStruct((NC, NROWS, CW), jnp.float32),
    mesh=_mesh,
    compiler_params=_sc_params,
    scratch_types=[
        pltpu.VMEM((ECH, K), jnp.int32),
        pltpu.VMEM((K, CW), jnp.float32),
        pltpu.VMEM((DRPT, CW), jnp.float32),
        pltpu.VMEM_SHARED((NROWS, CW), jnp.float32),
        pltpu.SemaphoreType.DMA,
    ],
)
def _deg_kernel(ei_hbm, ones_hbm, zeros_hbm, out_hbm, idx_v, ones_v, z_v,
                table, sem):
    c = lax.axis_index("c")
    s = lax.axis_index("s")

    idx_load = pltpu.async_copy(ei_hbm.at[c, s], idx_v, sem)
    pltpu.sync_copy(ones_hbm, ones_v)
    pltpu.sync_copy(zeros_hbm, z_v)

    pltpu.sync_copy(z_v, table.at[pl.ds(s * DRPT, DRPT)])
    idx_load.wait()
    plsc.subcore_barrier()

    def body(ch, carry):
        pltpu.sync_copy(ones_v, table.at[idx_v.at[ch]], add=True)
        return carry

    lax.fori_loop(0, ECH, body, 0)
    plsc.subcore_barrier()

    pltpu.sync_copy(table.at[pl.ds(s * DRPT, DRPT)], z_v)
    pltpu.sync_copy(z_v, out_hbm.at[c, pl.ds(s * DRPT, DRPT)])


@functools.partial(
    pl.kernel,
    out_type=jax.ShapeDtypeStruct((NROWS, D), jnp.float32),
    mesh=_mesh,
    compiler_params=_sc_params,
    scratch_types=(
        [pltpu.VMEM((ECH, K), jnp.int32)] * 2
        + [pltpu.VMEM((K, DH), jnp.float32)] * 5
        + [pltpu.VMEM_SHARED((NROWS, DH), jnp.float32)]
        + [pltpu.SemaphoreType.DMA] * 11
    ),
)
def _edge_kernel(hsv_hbm, ei_hbm, zeros_hbm, out_hbm, src_v, dst_v, *rest):
    bufs = rest[0:5]
    acc = rest[5]
    gsem = rest[6:11]
    ssem = rest[11:16]
    isem = rest[16]
    c = lax.axis_index("c")
    s = lax.axis_index("s")
    col = c * DH

    src_load = pltpu.async_copy(ei_hbm.at[0, s], src_v, isem)
    dst_load = pltpu.async_copy(ei_hbm.at[1, s], dst_v, isem)
    pltpu.sync_copy(zeros_hbm, bufs[0])

    base = s * RPT
    for j in range(4):
        pltpu.sync_copy(bufs[0], acc.at[pl.ds(base + j * K, K)])
    pltpu.sync_copy(bufs[0].at[pl.ds(0, RPT - 4 * K)],
                    acc.at[pl.ds(base + 4 * K, RPT - 4 * K)])
    src_load.wait()
    dst_load.wait()

    def xform(r, carry):
        for jj in range(K // 16):
            v = src_v[r, pl.ds(16 * jj, 16)]
            src_v[r, pl.ds(16 * jj, 16)] = v * 2 + c
        return carry

    lax.fori_loop(0, ECH, xform, 0)
    plsc.subcore_barrier()

    def issue_g(ch, b):
        pltpu.async_copy(hsv_hbm.at[src_v.at[ch]], bufs[b], gsem[b])

    def issue_s(ch, b):
        pltpu.async_copy(bufs[b], acc.at[dst_v.at[ch]], ssem[b], add=True)

    def gwait(b):
        pltpu.make_async_copy(hsv_hbm.at[src_v.at[0]], bufs[b], gsem[b]).wait()

    def swait(b):
        pltpu.make_async_copy(bufs[b], acc.at[dst_v.at[0]], ssem[b]).wait()

    for ch in range(4):
        issue_g(ch, ch)

    gwait(0); issue_s(0, 0); issue_g(4, 4)
    gwait(1); issue_s(1, 1); swait(0); issue_g(5, 0)
    gwait(2); issue_s(2, 2); swait(1); issue_g(6, 1)
    gwait(3); issue_s(3, 3); swait(2); issue_g(7, 2)
    gwait(4); issue_s(4, 4); swait(3); issue_g(8, 3)

    def body(i, carry):
        ch = 5 * i
        gwait(0); issue_s(ch, 0); swait(4); issue_g(ch + 4, 4)
        gwait(1); issue_s(ch + 1, 1); swait(0); issue_g(ch + 5, 0)
        gwait(2); issue_s(ch + 2, 2); swait(1); issue_g(ch + 6, 1)
        gwait(3); issue_s(ch + 3, 3); swait(2); issue_g(ch + 7, 2)
        gwait(4); issue_s(ch + 4, 4); swait(3); issue_g(ch + 8, 3)
        return carry

    lax.fori_loop(1, 30, body, 0)

    gwait(0); issue_s(150, 0); swait(4); issue_g(154, 4)
    gwait(1); issue_s(151, 1); swait(0); issue_g(155, 0)
    gwait(2); issue_s(152, 2); swait(1); issue_g(156, 1)
    gwait(3); issue_s(153, 3); swait(2)
    gwait(4); issue_s(154, 4); swait(3)
    gwait(0); issue_s(155, 0); swait(4)
    gwait(1); issue_s(156, 1); swait(0)
    swait(1)
    plsc.subcore_barrier()

    pltpu.sync_copy(acc.at[pl.ds(base, RPT)],
                    out_hbm.at[pl.ds(base, RPT), pl.ds(col, DH)])


_RB = 2000


def _prep_body(ds, dd, feat, so_ref, si_ref, fs_ref):
    so = lax.rsqrt(jnp.maximum(ds[0, :, 0:1], 1.0))
    si = lax.rsqrt(jnp.maximum(dd[0, :, 0:1], 1.0))
    so_ref[...] = so
    si_ref[...] = si
    fs_ref[...] = feat[...] * so


@jax.jit
def _prep(dcnt, feat):
    vec = pl.BlockSpec((_RB, 1), lambda i: (i, 0))
    cs = pl.BlockSpec((1, _RB, CW), lambda i: (0, i, 0))
    cd = pl.BlockSpec((1, _RB, CW), lambda i: (1, i, 0))
    return pl.pallas_call(
        _prep_body,
        grid=(N // _RB,),
        in_specs=[cs, cd, pl.BlockSpec((_RB, D), lambda i: (i, 0))],
        out_specs=[vec, vec, pl.BlockSpec((_RB, D), lambda i: (i, 0))],
        out_shape=[
            jax.ShapeDtypeStruct((N, 1), jnp.float32),
            jax.ShapeDtypeStruct((N, 1), jnp.float32),
            jax.ShapeDtypeStruct((NROWS, D), jnp.float32),
        ],
    )(dcnt, dcnt, feat)


def _dense_body(want_h, parts, si, so, w, a, out_ref, pool_ref):
    i = pl.program_id(0)
    agg = parts[...] * si[...]
    out = jnp.dot(agg, w[...], preferred_element_type=jnp.float32)
    aa = a[0, 0]
    h = jnp.where(out >= 0.0, out, aa * out)
    if want_h:
        out_ref[...] = h
    else:
        out_ref[...] = h * so[...]

    @pl.when(i == 0)
    def _():
        pool_ref[...] = jnp.zeros_like(pool_ref)

    pool_ref[...] += jnp.sum(h, axis=0, keepdims=True)


@functools.partial(jax.jit, static_argnums=0)
def _dense(want_h, parts, si, so, w, a):
    vec = pl.BlockSpec((_RB, 1), lambda i: (i, 0))
    if want_h:
        out_spec = pl.BlockSpec((_RB, D), lambda i: (i, 0))
        out_shape = jax.ShapeDtypeStruct((N, D), jnp.float32)
    else:
        out_spec = pl.BlockSpec((_RB, D), lambda i: (i, 0))
        out_shape = jax.ShapeDtypeStruct((NROWS, D), jnp.float32)
    return pl.pallas_call(
        functools.partial(_dense_body, want_h),
        grid=(N // _RB,),
        in_specs=[
            pl.BlockSpec((_RB, D), lambda i: (i, 0)),
            vec, vec,
            pl.BlockSpec((D, D), lambda i: (0, 0)),
            pl.BlockSpec(memory_space=pltpu.SMEM),
        ],
        out_specs=[out_spec, pl.BlockSpec((1, D), lambda i: (0, 0))],
        out_shape=[out_shape, jax.ShapeDtypeStruct((1, D), jnp.float32)],
    )(parts, si, so, w, a)


def kernel(feat, edge_index, W0, W1, a0, a1):
    ei_p = jnp.pad(edge_index, ((0, 0), (0, EPAD - E)),
                   constant_values=DUMP).reshape(2, NS, ECH, K)
    ei_p = jax.lax.optimization_barrier(ei_p)

    ones_cw = jnp.ones((K, CW), jnp.float32)
    zeros_cw = jnp.zeros((DRPT, CW), jnp.float32)
    zeros_kd = jnp.zeros((K, DH), jnp.float32)

    dcnt = _deg_kernel(ei_p, ones_cw, zeros_cw)
    s_out, s_in, fs2 = _prep(dcnt, feat)

    a0_2d = a0.reshape(1, 1)
    a1_2d = a1.reshape(1, 1)

    parts1 = _edge_kernel(fs2.reshape(2 * NROWS, DH), ei_p, zeros_kd)
    hs2, pool1 = _dense(False, parts1, s_in, s_out, W0, a0_2d)

    parts2 = _edge_kernel(hs2.reshape(2 * NROWS, DH), ei_p, zeros_kd)
    h2, pool2 = _dense(True, parts2, s_in, s_out, W1, a1_2d)

    hg = jnp.concatenate([pool1, pool2], axis=-1)
    return (h2, hg)
